# baseline (speedup 1.0000x reference)
import functools

import jax
import jax.numpy as jnp
from jax import lax
from jax.experimental import pallas as pl
from jax.experimental.pallas import tpu as pltpu
from jax.experimental.pallas import tpu_sc as plsc

BATCH = 16384
INPUT_DIM = 1000000
L = 26
NC = 2
NS = 16
NW = NC * NS
RPT = BATCH // NW
IPT = RPT * L
L0 = 13
L1 = L - L0


def _sc_kernel(xt_hbm, w_hbm, b_hbm, out_hbm, x_v, vals_v, b_v, acc_v,
               sem_a, sem_b, sem_g):
    wid = lax.axis_index("s") * NC + lax.axis_index("c")
    col = pl.ds(wid * RPT, RPT)

    ha = [
        pltpu.async_copy(xt_hbm.at[j].at[col], x_v.at[pl.ds(j * RPT, RPT)],
                         sem_a)
        for j in range(L0)
    ]
    hb = [
        pltpu.async_copy(xt_hbm.at[j].at[col], x_v.at[pl.ds(j * RPT, RPT)],
                         sem_b)
        for j in range(L0, L)
    ]
    pltpu.sync_copy(b_hbm, b_v)
    for h in ha:
        h.wait()
    g0 = pltpu.async_copy(
        w_hbm.at[0].at[x_v.at[pl.ds(0, L0 * RPT)]],
        vals_v.at[pl.ds(0, L0 * RPT)], sem_g)
    for h in hb:
        h.wait()
    g0.wait()
    g1 = pltpu.async_copy(
        w_hbm.at[0].at[x_v.at[pl.ds(L0 * RPT, L1 * RPT)]],
        vals_v.at[pl.ds(L0 * RPT, L1 * RPT)], sem_g)

    def body0(g, _):
        base = g * 16
        acc = jnp.zeros((16,), jnp.float32)
        for j in range(L0):
            acc = acc + vals_v[pl.ds(j * RPT + base, 16)]
        acc_v[pl.ds(base, 16)] = acc
        return 0

    lax.fori_loop(0, RPT // 16, body0, 0)

    g1.wait()
    bias = b_v[...]

    def body1(g, _):
        base = g * 16
        acc = acc_v[pl.ds(base, 16)]
        for j in range(L0, L):
            acc = acc + vals_v[pl.ds(j * RPT + base, 16)]
        z = (acc + bias) * 2.0
        acc_v[pl.ds(base, 16)] = 1.0 / (1.0 + jnp.exp(-z))
        return 0

    lax.fori_loop(0, RPT // 16, body1, 0)

    pltpu.sync_copy(acc_v, out_hbm.at[pl.ds(wid * RPT, RPT)])


@jax.jit
def _run(xt, w_flat, b16):
    mesh = plsc.VectorSubcoreMesh(core_axis_name="c", subcore_axis_name="s")
    f = functools.partial(
        pl.kernel,
        mesh=mesh,
        out_type=jax.ShapeDtypeStruct((BATCH,), jnp.float32),
        scratch_types=[
            pltpu.VMEM((IPT,), jnp.int32),
            pltpu.VMEM((IPT,), jnp.float32),
            pltpu.VMEM((16,), jnp.float32),
            pltpu.VMEM((RPT,), jnp.float32),
            pltpu.SemaphoreType.DMA,
            pltpu.SemaphoreType.DMA,
            pltpu.SemaphoreType.DMA,
        ],
    )(_sc_kernel)
    return f(xt, w_flat, b16)


def kernel(x, w, b):
    xt = x.T
    w_flat = w.reshape(1, INPUT_DIM)
    b16 = jnp.broadcast_to(b, (16,))
    out = _run(xt, w_flat, b16)
    return out.reshape(BATCH, 1)


# speedup vs baseline: 1.0104x; 1.0104x over previous
import functools

import jax
import jax.numpy as jnp
from jax import lax
from jax.experimental import pallas as pl
from jax.experimental.pallas import tpu as pltpu
from jax.experimental.pallas import tpu_sc as plsc

BATCH = 16384
INPUT_DIM = 1000000
L = 26
NC = 2
NS = 16
NW = NC * NS
RPT = BATCH // NW
IPT = RPT * L
L0 = 13
L1 = L - L0


def _sc_kernel(xt_hbm, w_hbm, b_hbm, out_hbm, x_v, vals_v, b_v, acc_v,
               sem_a, sem_b, sem_g):
    wid = lax.axis_index("s") * NC + lax.axis_index("c")
    col = pl.ds(wid * RPT, RPT)

    ha = [
        pltpu.async_copy(xt_hbm.at[j].at[col], x_v.at[pl.ds(j * RPT, RPT)],
                         sem_a)
        for j in range(L0)
    ]
    hb = [
        pltpu.async_copy(xt_hbm.at[j].at[col], x_v.at[pl.ds(j * RPT, RPT)],
                         sem_b)
        for j in range(L0, L)
    ]
    pltpu.sync_copy(b_hbm, b_v)
    for h in ha:
        h.wait()
    g0 = pltpu.async_copy(
        w_hbm.at[0].at[x_v.at[pl.ds(0, L0 * RPT)]],
        vals_v.at[pl.ds(0, L0 * RPT)], sem_g)
    for h in hb:
        h.wait()
    g1 = pltpu.async_copy(
        w_hbm.at[0].at[x_v.at[pl.ds(L0 * RPT, L1 * RPT)]],
        vals_v.at[pl.ds(L0 * RPT, L1 * RPT)], sem_b)
    g0.wait()

    def body0(g, _):
        base = g * 16
        acc = jnp.zeros((16,), jnp.float32)
        for j in range(L0):
            acc = acc + vals_v[pl.ds(j * RPT + base, 16)]
        acc_v[pl.ds(base, 16)] = acc
        return 0

    lax.fori_loop(0, RPT // 16, body0, 0)

    g1.wait()
    bias = b_v[...]

    def body1(g, _):
        base = g * 16
        acc = acc_v[pl.ds(base, 16)]
        for j in range(L0, L):
            acc = acc + vals_v[pl.ds(j * RPT + base, 16)]
        z = (acc + bias) * 2.0
        acc_v[pl.ds(base, 16)] = 1.0 / (1.0 + jnp.exp(-z))
        return 0

    lax.fori_loop(0, RPT // 16, body1, 0)

    pltpu.sync_copy(acc_v, out_hbm.at[pl.ds(wid * RPT, RPT)])


@jax.jit
def _run(xt, w_flat, b16):
    mesh = plsc.VectorSubcoreMesh(core_axis_name="c", subcore_axis_name="s")
    f = functools.partial(
        pl.kernel,
        mesh=mesh,
        out_type=jax.ShapeDtypeStruct((BATCH,), jnp.float32),
        scratch_types=[
            pltpu.VMEM((IPT,), jnp.int32),
            pltpu.VMEM((IPT,), jnp.float32),
            pltpu.VMEM((16,), jnp.float32),
            pltpu.VMEM((RPT,), jnp.float32),
            pltpu.SemaphoreType.DMA,
            pltpu.SemaphoreType.DMA,
            pltpu.SemaphoreType.DMA,
        ],
    )(_sc_kernel)
    return f(xt, w_flat, b16)


def kernel(x, w, b):
    xt = x.T
    w_flat = w.reshape(1, INPUT_DIM)
    b16 = jnp.broadcast_to(b, (16,))
    out = _run(xt, w_flat, b16)
    return out.reshape(BATCH, 1)
